# baseline (speedup 1.0000x reference)
import jax
import jax.numpy as jnp
from jax.experimental import pallas as pl
from jax.experimental.pallas import tpu as pltpu

_ROWS = 8192
_COLS = 4096
_MAXCHUNK = 512
_NBUF = 3

_SCH = [128, 128, 256] + [512] * 14 + [256, 128, 128]
assert sum(_SCH) == _ROWS
_OFFS = [sum(_SCH[:i]) for i in range(len(_SCH))]
_NCH = len(_SCH)


def _gelu(x):
    return 0.5 * x * (1.0 + jax.lax.erf(x * 0.7071067811865476))


def _body(x_hbm, o_hbm, inbuf, outbuf, insem, outsem):
    def in_copy(i):
        slot = i % _NBUF
        return pltpu.make_async_copy(
            x_hbm.at[pl.ds(_OFFS[i], _SCH[i]), :],
            inbuf.at[slot, pl.ds(0, _SCH[i]), :],
            insem.at[slot],
        )

    def out_copy(i):
        slot = i % _NBUF
        return pltpu.make_async_copy(
            outbuf.at[slot, pl.ds(0, _SCH[i]), :],
            o_hbm.at[pl.ds(_OFFS[i], _SCH[i]), :],
            outsem.at[slot],
        )

    for b in range(_NBUF):
        in_copy(b).start()

    for i in range(_NCH):
        slot = i % _NBUF
        in_copy(i).wait()
        if i >= _NBUF:
            out_copy(i - _NBUF).wait()
        n = _SCH[i]
        outbuf[slot, :n, :] = _gelu(inbuf[slot, :n, :])
        out_copy(i).start()
        if i + _NBUF < _NCH:
            in_copy(i + _NBUF).start()

    for i in range(_NCH - _NBUF, _NCH):
        out_copy(i).wait()


def kernel(input):
    x = input.reshape(_ROWS, _COLS)
    out = pl.pallas_call(
        _body,
        out_shape=jax.ShapeDtypeStruct((_ROWS, _COLS), jnp.float32),
        in_specs=[pl.BlockSpec(memory_space=pl.ANY)],
        out_specs=pl.BlockSpec(memory_space=pl.ANY),
        scratch_shapes=[
            pltpu.VMEM((_NBUF, _MAXCHUNK, _COLS), jnp.float32),
            pltpu.VMEM((_NBUF, _MAXCHUNK, _COLS), jnp.float32),
            pltpu.SemaphoreType.DMA((_NBUF,)),
            pltpu.SemaphoreType.DMA((_NBUF,)),
        ],
    )(x)
    return out.reshape(input.shape)


# speedup vs baseline: 1.0023x; 1.0023x over previous
import jax
import jax.numpy as jnp
from jax.experimental import pallas as pl
from jax.experimental.pallas import tpu as pltpu

_ROWS = 8192
_COLS = 4096
_MAXCHUNK = 512
_NBUF = 3

_SCH = [128, 384] + [512] * 14 + [384, 128]
assert sum(_SCH) == _ROWS
_OFFS = [sum(_SCH[:i]) for i in range(len(_SCH))]
_NCH = len(_SCH)


def _gelu(x):
    return 0.5 * x * (1.0 + jax.lax.erf(x * 0.7071067811865476))


def _body(x_hbm, o_hbm, inbuf, outbuf, insem, outsem):
    def in_copy(i):
        slot = i % _NBUF
        return pltpu.make_async_copy(
            x_hbm.at[pl.ds(_OFFS[i], _SCH[i]), :],
            inbuf.at[slot, pl.ds(0, _SCH[i]), :],
            insem.at[slot],
        )

    def out_copy(i):
        slot = i % _NBUF
        return pltpu.make_async_copy(
            outbuf.at[slot, pl.ds(0, _SCH[i]), :],
            o_hbm.at[pl.ds(_OFFS[i], _SCH[i]), :],
            outsem.at[slot],
        )

    for b in range(_NBUF):
        in_copy(b).start()

    for i in range(_NCH):
        slot = i % _NBUF
        in_copy(i).wait()
        if i >= _NBUF:
            out_copy(i - _NBUF).wait()
        n = _SCH[i]
        outbuf[slot, :n, :] = _gelu(inbuf[slot, :n, :])
        out_copy(i).start()
        if i + _NBUF < _NCH:
            in_copy(i + _NBUF).start()

    for i in range(_NCH - _NBUF, _NCH):
        out_copy(i).wait()


def kernel(input):
    x = input.reshape(_ROWS, _COLS)
    out = pl.pallas_call(
        _body,
        out_shape=jax.ShapeDtypeStruct((_ROWS, _COLS), jnp.float32),
        in_specs=[pl.BlockSpec(memory_space=pl.ANY)],
        out_specs=pl.BlockSpec(memory_space=pl.ANY),
        scratch_shapes=[
            pltpu.VMEM((_NBUF, _MAXCHUNK, _COLS), jnp.float32),
            pltpu.VMEM((_NBUF, _MAXCHUNK, _COLS), jnp.float32),
            pltpu.SemaphoreType.DMA((_NBUF,)),
            pltpu.SemaphoreType.DMA((_NBUF,)),
        ],
    )(x)
    return out.reshape(input.shape)
